# Initial kernel scaffold; baseline (speedup 1.0000x reference)
import jax
import jax.numpy as jnp
from jax.experimental import pallas as pl


def kernel(id2feat, nodes, adj_neighs, dis_neighs, Wm_w, Wm_b, Wq_w, Wq_b, Wk_w, Wk_b, Wv_w, Wv_b, WC_w, WC_b):
    raise NotImplementedError("write your pallas kernel here")


# speedup vs baseline: 2.7010x; 2.7010x over previous
import functools

import jax
import jax.numpy as jnp
from jax import lax
from jax.experimental import pallas as pl
from jax.experimental.pallas import tpu as pltpu
from jax.experimental.pallas import tpu_sc as plsc

N, FD, ED, S, B = 10000, 128, 64, 32, 10000
NC, NS = 2, 16
NW = NC * NS
CH = 128
BPC = CH // S
NCH = (B * S) // CH
QCH = 80
NQCH = B // QCH
BB = 200
PROJ_BLK = 1000


def _proj_body(x_ref, wm_ref, wkv_ref, wq_ref, bm_ref, bkv_ref, bq_ref,
               pm_ref, pkv_ref, pq_ref):
    x = x_ref[:]
    pm_ref[:] = (jnp.dot(x, wm_ref[:], preferred_element_type=jnp.float32)
                 + bm_ref[:]) * (1.0 / S)
    pkv_ref[:] = jnp.dot(x, wkv_ref[:], preferred_element_type=jnp.float32) + bkv_ref[:]
    pq_ref[:] = jnp.dot(x, wq_ref[:], preferred_element_type=jnp.float32) + bq_ref[:]


def _project(id2feat, wm, wkv, wq, bm, bkv, bq):
    nblk = N // PROJ_BLK
    full = lambda i: (0, 0)
    return pl.pallas_call(
        _proj_body,
        grid=(nblk,),
        in_specs=[
            pl.BlockSpec((PROJ_BLK, FD), lambda i: (i, 0)),
            pl.BlockSpec((FD, ED), full),
            pl.BlockSpec((FD, 2 * ED), full),
            pl.BlockSpec((FD, ED), full),
            pl.BlockSpec((1, ED), full),
            pl.BlockSpec((1, 2 * ED), full),
            pl.BlockSpec((1, ED), full),
        ],
        out_specs=[
            pl.BlockSpec((PROJ_BLK, ED), lambda i: (i, 0)),
            pl.BlockSpec((PROJ_BLK, 2 * ED), lambda i: (i, 0)),
            pl.BlockSpec((PROJ_BLK, ED), lambda i: (i, 0)),
        ],
        out_shape=[
            jax.ShapeDtypeStruct((N, ED), jnp.float32),
            jax.ShapeDtypeStruct((N, 2 * ED), jnp.float32),
            jax.ShapeDtypeStruct((N, ED), jnp.float32),
        ],
    )(id2feat, wm, wkv, wq, bm, bkv, bq)


def _make_sc_mean_q():
    mesh = plsc.VectorSubcoreMesh(core_axis_name="c", subcore_axis_name="s")

    @functools.partial(
        pl.kernel,
        mesh=mesh,
        out_type=[
            jax.ShapeDtypeStruct((B, ED), jnp.float32),
            jax.ShapeDtypeStruct((B, ED), jnp.float32),
        ],
        scratch_types=[
            pltpu.VMEM((CH,), jnp.int32),
            pltpu.VMEM((QCH,), jnp.int32),
            pltpu.VMEM((CH,), jnp.int32),
            pltpu.VMEM((CH, ED), jnp.float32),
            pltpu.VMEM((QCH, ED), jnp.float32),
            pltpu.VMEM((BPC, ED), jnp.float32),
            pltpu.VMEM((BPC, ED), jnp.float32),
            pltpu.VMEM_SHARED((NS * BPC, ED), jnp.float32),
            pltpu.SemaphoreType.DMA,
        ],
        compiler_params=pltpu.CompilerParams(use_tc_tiling_on_sc=False),
    )
    def sc_mean_q(pm, pq, dis_i, nod_i, dst_all, zeros_h, ctx, qg,
                  idx_d, idx_q, dst_v, rows_m, rows_q, zrow, ctx_b, acc, sem):
        sid = lax.axis_index("s")
        w = lax.axis_index("c") * NS + sid

        pltpu.sync_copy(dst_all.at[sid], dst_v)
        pltpu.sync_copy(zeros_h, zrow)

        def dis_iter(k, carry):
            g = w + NW * k

            @pl.when(g < NCH)
            def _():
                pltpu.sync_copy(dis_i.at[g], idx_d)
                pltpu.async_copy(pm.at[idx_d], rows_m, sem).wait()
                pltpu.sync_copy(zrow, acc.at[pl.ds(sid * BPC, BPC)])
                pltpu.sync_copy(rows_m, acc.at[dst_v], add=True)
                pltpu.sync_copy(acc.at[pl.ds(sid * BPC, BPC)], ctx_b)
                pltpu.sync_copy(ctx_b, ctx.at[pl.ds(g * BPC, BPC)])
            return carry

        def q_iter(k, carry):
            g = w + NW * k

            @pl.when(g < NQCH)
            def _():
                pltpu.sync_copy(nod_i.at[g], idx_q)
                pltpu.async_copy(pq.at[idx_q], rows_q, sem).wait()
                pltpu.sync_copy(rows_q, qg.at[pl.ds(g * QCH, QCH)])
            return carry

        lax.fori_loop(0, (NCH + NW - 1) // NW, dis_iter, 0)
        lax.fori_loop(0, (NQCH + NW - 1) // NW, q_iter, 0)

    return sc_mean_q


_sc_mean_q = _make_sc_mean_q()


def _make_sc_kv():
    mesh = plsc.VectorSubcoreMesh(core_axis_name="c", subcore_axis_name="s")

    @functools.partial(
        pl.kernel,
        mesh=mesh,
        out_type=jax.ShapeDtypeStruct((B * S, 2 * ED), jnp.float32),
        scratch_types=[
            pltpu.VMEM((CH,), jnp.int32),
            pltpu.VMEM((CH, 2 * ED), jnp.float32),
            pltpu.SemaphoreType.DMA,
        ],
    )
    def sc_kv(pkv, adj_i, kvg, idx_a, rows_kv, sem):
        w = lax.axis_index("c") * NS + lax.axis_index("s")

        def adj_iter(k, carry):
            g = w + NW * k

            @pl.when(g < NCH)
            def _():
                pltpu.sync_copy(adj_i.at[g], idx_a)
                pltpu.async_copy(pkv.at[idx_a], rows_kv, sem).wait()
                pltpu.sync_copy(rows_kv, kvg.at[pl.ds(g * CH, CH)])
            return carry

        lax.fori_loop(0, (NCH + NW - 1) // NW, adj_iter, 0)

    return sc_kv


_sc_kv = _make_sc_kv()


def _attn_body(kvg_ref, ctx_ref, qg_ref, wc_ref, bc_ref, out_ref):
    kv = kvg_ref[:].reshape(BB, S, 2 * ED)
    k3 = kv[:, :, :ED]
    v3 = kv[:, :, ED:]
    q = qg_ref[:]
    logits = jnp.sum(k3 * q[:, None, :], axis=-1)
    m = jnp.max(logits, axis=-1, keepdims=True)
    e = jnp.exp(logits - m)
    attn = e / jnp.sum(e, axis=-1, keepdims=True)
    mix = jnp.sum(v3 * attn[:, :, None], axis=1)
    comb = jnp.concatenate([mix, ctx_ref[:]], axis=-1)
    comb = jnp.tanh(jnp.dot(comb, wc_ref[:], preferred_element_type=jnp.float32)
                    + bc_ref[:])
    nrm = jnp.sqrt(jnp.sum(comb * comb, axis=-1, keepdims=True))
    out_ref[:] = comb / jnp.maximum(nrm, 1e-12)


def _attention(kvg, ctx, qg, wc, bc):
    nblk = B // BB
    return pl.pallas_call(
        _attn_body,
        grid=(nblk,),
        in_specs=[
            pl.BlockSpec((BB * S, 2 * ED), lambda i: (i, 0)),
            pl.BlockSpec((BB, ED), lambda i: (i, 0)),
            pl.BlockSpec((BB, ED), lambda i: (i, 0)),
            pl.BlockSpec((2 * ED, ED), lambda i: (0, 0)),
            pl.BlockSpec((1, ED), lambda i: (0, 0)),
        ],
        out_specs=pl.BlockSpec((BB, ED), lambda i: (i, 0)),
        out_shape=jax.ShapeDtypeStruct((B, ED), jnp.float32),
    )(kvg, ctx, qg, wc, bc)


def kernel(id2feat, nodes, adj_neighs, dis_neighs, Wm_w, Wm_b, Wq_w, Wq_b,
           Wk_w, Wk_b, Wv_w, Wv_b, WC_w, WC_b):
    wkv = jnp.concatenate([Wk_w, Wv_w], axis=1)
    bkv = jnp.concatenate([Wk_b, Wv_b])[None, :]
    pm, pkv, pq = _project(id2feat, Wm_w, wkv, Wq_w,
                           Wm_b[None, :], bkv, Wq_b[None, :])
    adj2 = adj_neighs.astype(jnp.int32).reshape(NCH, CH)
    dis2 = dis_neighs.astype(jnp.int32).reshape(NCH, CH)
    nod2 = nodes.astype(jnp.int32).reshape(NQCH, QCH)
    dst_all = (jnp.arange(NS, dtype=jnp.int32)[:, None] * BPC
               + (jnp.arange(CH, dtype=jnp.int32) // S)[None, :])
    zeros_h = jnp.zeros((BPC, ED), jnp.float32)
    ctx, qg = _sc_mean_q(pm, pq, dis2, nod2, dst_all, zeros_h)
    kvg = _sc_kv(pkv, adj2)
    return _attention(kvg, ctx, qg, WC_w, WC_b[None, :])


# speedup vs baseline: 3.3244x; 1.2308x over previous
import functools

import jax
import jax.numpy as jnp
from jax import lax
from jax.experimental import pallas as pl
from jax.experimental.pallas import tpu as pltpu
from jax.experimental.pallas import tpu_sc as plsc

N, FD, ED, S, B = 10000, 128, 64, 32, 10000
NC, NS = 2, 16
NW = NC * NS
CH = 128
NCH = (B * S) // CH
QCH = 80
NQCH = B // QCH
NBUF = 4
QBUF = 2
EPW = 80
NCHP = NW * EPW
QPW = 8
NQCHP = 128
BB = 200
PROJ_BLK = 1000


def _proj_body(x_ref, wkv_ref, wmq_ref, bkv_ref, bmq_ref, pkv_ref, pmq_ref):
    x = x_ref[:]
    pkv_ref[:] = jnp.dot(x, wkv_ref[:], preferred_element_type=jnp.float32) + bkv_ref[:]
    pmq_ref[:] = jnp.dot(x, wmq_ref[:], preferred_element_type=jnp.float32) + bmq_ref[:]


def _project(id2feat, wkv, wmq, bkv, bmq):
    nblk = N // PROJ_BLK
    full = lambda i: (0, 0)
    return pl.pallas_call(
        _proj_body,
        grid=(nblk,),
        in_specs=[
            pl.BlockSpec((PROJ_BLK, FD), lambda i: (i, 0)),
            pl.BlockSpec((FD, 2 * ED), full),
            pl.BlockSpec((FD, 2 * ED), full),
            pl.BlockSpec((1, 2 * ED), full),
            pl.BlockSpec((1, 2 * ED), full),
        ],
        out_specs=[
            pl.BlockSpec((PROJ_BLK, 2 * ED), lambda i: (i, 0)),
            pl.BlockSpec((PROJ_BLK, 2 * ED), lambda i: (i, 0)),
        ],
        out_shape=[
            jax.ShapeDtypeStruct((N, 2 * ED), jnp.float32),
            jax.ShapeDtypeStruct((N, 2 * ED), jnp.float32),
        ],
    )(id2feat, wkv, wmq, bkv, bmq)


def _make_sc_gather():
    mesh = plsc.VectorSubcoreMesh(core_axis_name="c", subcore_axis_name="s")

    @functools.partial(
        pl.kernel,
        mesh=mesh,
        out_type=[
            jax.ShapeDtypeStruct((B * S, 2 * ED), jnp.float32),
            jax.ShapeDtypeStruct((B * S, 2 * ED), jnp.float32),
            jax.ShapeDtypeStruct((B, 2 * ED), jnp.float32),
        ],
        scratch_types=(
            [pltpu.VMEM((EPW, CH), jnp.int32),
             pltpu.VMEM((QPW, QCH), jnp.int32)]
            + [pltpu.VMEM((CH, 2 * ED), jnp.float32) for _ in range(NBUF)]
            + [pltpu.VMEM((QCH, 2 * ED), jnp.float32) for _ in range(QBUF)]
            + [pltpu.SemaphoreType.DMA for _ in range(2 * NBUF)]
        ),
    )
    def sc_gather(pkv, pmq, adj_i, dis_i, nod_i, kvg, mqg, qg, *scratch):
        idx_all, idx_q = scratch[0], scratch[1]
        rows = scratch[2:2 + NBUF]
        rows_q = scratch[2 + NBUF:2 + NBUF + QBUF]
        sem_g = scratch[2 + NBUF + QBUF:2 + NBUF + QBUF + NBUF]
        sem_w = scratch[2 + NBUF + QBUF + NBUF:]
        w = lax.axis_index("c") * NS + lax.axis_index("s")

        def edge_job(tab, idx2d, out):
            lo = w * EPW
            cnt = jnp.clip(NCH - lo, 0, EPW)
            pltpu.sync_copy(idx2d.at[pl.ds(lo, EPW)], idx_all)
            ngrp = EPW // NBUF
            kmax = ngrp * NBUF

            def grp(it, carry):
                for b in range(NBUF):
                    k = it * NBUF + b

                    @pl.when(jnp.logical_and(k >= NBUF, k - NBUF < cnt))
                    def _():
                        pltpu.make_async_copy(
                            out.at[pl.ds(0, CH)], rows[b], sem_w[b]).wait()

                    @pl.when(k < cnt)
                    def _():
                        pltpu.async_copy(tab.at[idx_all.at[k]], rows[b], sem_g[b])
                for b in range(NBUF):
                    k = it * NBUF + b

                    @pl.when(k < cnt)
                    def _():
                        pltpu.make_async_copy(
                            tab.at[idx_all.at[k]], rows[b], sem_g[b]).wait()
                        pltpu.async_copy(
                            rows[b], out.at[pl.ds((lo + k) * CH, CH)], sem_w[b])
                return carry

            lax.fori_loop(0, ngrp, grp, 0)
            for b in range(NBUF):
                @pl.when(kmax - NBUF + b < cnt)
                def _():
                    pltpu.make_async_copy(
                        out.at[pl.ds(0, CH)], rows[b], sem_w[b]).wait()

        def q_job():
            lo = jnp.minimum(w * QPW, NQCHP - QPW)
            cnt = jnp.clip(NQCH - w * QPW, 0, QPW)
            pltpu.sync_copy(nod_i.at[pl.ds(lo, QPW)], idx_q)
            ngrp = QPW // QBUF
            kmax = ngrp * QBUF

            def grp(it, carry):
                for b in range(QBUF):
                    k = it * QBUF + b

                    @pl.when(jnp.logical_and(k >= QBUF, k - QBUF < cnt))
                    def _():
                        pltpu.make_async_copy(
                            qg.at[pl.ds(0, QCH)], rows_q[b], sem_w[b]).wait()

                    @pl.when(k < cnt)
                    def _():
                        pltpu.async_copy(pmq.at[idx_q.at[k]], rows_q[b], sem_g[b])
                for b in range(QBUF):
                    k = it * QBUF + b

                    @pl.when(k < cnt)
                    def _():
                        pltpu.make_async_copy(
                            pmq.at[idx_q.at[k]], rows_q[b], sem_g[b]).wait()
                        pltpu.async_copy(
                            rows_q[b], qg.at[pl.ds((lo + k) * QCH, QCH)], sem_w[b])
                return carry

            lax.fori_loop(0, ngrp, grp, 0)
            for b in range(QBUF):
                @pl.when(kmax - QBUF + b < cnt)
                def _():
                    pltpu.make_async_copy(
                        qg.at[pl.ds(0, QCH)], rows_q[b], sem_w[b]).wait()

        edge_job(pkv, adj_i, kvg)
        edge_job(pmq, dis_i, mqg)
        q_job()

    return sc_gather


_sc_gather = _make_sc_gather()


def _attn_body(kvg_ref, mg_ref, qg_ref, wc_ref, bc_ref, out_ref):
    kv = kvg_ref[:].reshape(BB, S, 2 * ED)
    k3 = kv[:, :, :ED]
    v3 = kv[:, :, ED:]
    q = qg_ref[:, ED:]
    logits = jnp.sum(k3 * q[:, None, :], axis=-1)
    m = jnp.max(logits, axis=-1, keepdims=True)
    e = jnp.exp(logits - m)
    attn = e / jnp.sum(e, axis=-1, keepdims=True)
    mix = jnp.sum(v3 * attn[:, :, None], axis=1)
    ctx = jnp.sum(mg_ref[:].reshape(BB, S, 2 * ED)[:, :, :ED], axis=1)
    comb = jnp.concatenate([mix, ctx], axis=-1)
    comb = jnp.tanh(jnp.dot(comb, wc_ref[:], preferred_element_type=jnp.float32)
                    + bc_ref[:])
    nrm = jnp.sqrt(jnp.sum(comb * comb, axis=-1, keepdims=True))
    out_ref[:] = comb / jnp.maximum(nrm, 1e-12)


def _attention(kvg, mqg, qg, wc, bc):
    nblk = B // BB
    return pl.pallas_call(
        _attn_body,
        grid=(nblk,),
        in_specs=[
            pl.BlockSpec((BB * S, 2 * ED), lambda i: (i, 0)),
            pl.BlockSpec((BB * S, 2 * ED), lambda i: (i, 0)),
            pl.BlockSpec((BB, 2 * ED), lambda i: (i, 0)),
            pl.BlockSpec((2 * ED, ED), lambda i: (0, 0)),
            pl.BlockSpec((1, ED), lambda i: (0, 0)),
        ],
        out_specs=pl.BlockSpec((BB, ED), lambda i: (i, 0)),
        out_shape=jax.ShapeDtypeStruct((B, ED), jnp.float32),
    )(kvg, mqg, qg, wc, bc)


def kernel(id2feat, nodes, adj_neighs, dis_neighs, Wm_w, Wm_b, Wq_w, Wq_b,
           Wk_w, Wk_b, Wv_w, Wv_b, WC_w, WC_b):
    wkv = jnp.concatenate([Wk_w, Wv_w], axis=1)
    bkv = jnp.concatenate([Wk_b, Wv_b])[None, :]
    wmq = jnp.concatenate([Wm_w / S, Wq_w], axis=1)
    bmq = jnp.concatenate([Wm_b / S, Wq_b])[None, :]
    pkv, pmq = _project(id2feat, wkv, wmq, bkv, bmq)
    epad = jnp.zeros((NCHP - NCH, CH), jnp.int32)
    adj2 = jnp.concatenate([adj_neighs.astype(jnp.int32).reshape(NCH, CH), epad])
    dis2 = jnp.concatenate([dis_neighs.astype(jnp.int32).reshape(NCH, CH), epad])
    nod2 = jnp.concatenate([nodes.astype(jnp.int32).reshape(NQCH, QCH),
                            jnp.zeros((NQCHP - NQCH, QCH), jnp.int32)])
    kvg, mqg, qg = _sc_gather(pkv, pmq, adj2, dis2, nod2)
    return _attention(kvg, mqg, qg, WC_w, WC_b[None, :])


# speedup vs baseline: 3.7919x; 1.1406x over previous
import functools

import jax
import jax.numpy as jnp
from jax import lax
from jax.experimental import pallas as pl
from jax.experimental.pallas import tpu as pltpu
from jax.experimental.pallas import tpu_sc as plsc

N, FD, ED, S, B = 10000, 128, 64, 32, 10000
NC, NS = 2, 16
NW = NC * NS
CH = 128
BPC = CH // S
NCH = (B * S) // CH
QCH = 80
NQCH = B // QCH
NBUF = 4
QBUF = 2
EPW = 80
NCHP = NW * EPW
QPW = 8
NQCHP = 128
GB = NBUF * BPC
BB = 200
PROJ_BLK = 1000


def _proj_body(x_ref, wkv_ref, wm_ref, wq_ref, bkv_ref, bm_ref, bq_ref,
               pkv_ref, pm_ref, pq_ref):
    x = x_ref[:]
    pkv_ref[:] = jnp.dot(x, wkv_ref[:], preferred_element_type=jnp.float32) + bkv_ref[:]
    pm_ref[:] = (jnp.dot(x, wm_ref[:], preferred_element_type=jnp.float32)
                 + bm_ref[:]) * (1.0 / S)
    pq_ref[:] = jnp.dot(x, wq_ref[:], preferred_element_type=jnp.float32) + bq_ref[:]


def _project(id2feat, wkv, wm, wq, bkv, bm, bq):
    nblk = N // PROJ_BLK
    full = lambda i: (0, 0)
    return pl.pallas_call(
        _proj_body,
        grid=(nblk,),
        in_specs=[
            pl.BlockSpec((PROJ_BLK, FD), lambda i: (i, 0)),
            pl.BlockSpec((FD, 2 * ED), full),
            pl.BlockSpec((FD, ED), full),
            pl.BlockSpec((FD, ED), full),
            pl.BlockSpec((1, 2 * ED), full),
            pl.BlockSpec((1, ED), full),
            pl.BlockSpec((1, ED), full),
        ],
        out_specs=[
            pl.BlockSpec((PROJ_BLK, 2 * ED), lambda i: (i, 0)),
            pl.BlockSpec((PROJ_BLK, ED), lambda i: (i, 0)),
            pl.BlockSpec((PROJ_BLK, ED), lambda i: (i, 0)),
        ],
        out_shape=[
            jax.ShapeDtypeStruct((N, 2 * ED), jnp.float32),
            jax.ShapeDtypeStruct((N, ED), jnp.float32),
            jax.ShapeDtypeStruct((N, ED), jnp.float32),
        ],
    )(id2feat, wkv, wm, wq, bkv, bm, bq)


def _make_sc_kv():
    mesh = plsc.VectorSubcoreMesh(core_axis_name="c", subcore_axis_name="s")

    @functools.partial(
        pl.kernel,
        mesh=mesh,
        out_type=jax.ShapeDtypeStruct((B * S, 2 * ED), jnp.float32),
        scratch_types=(
            [pltpu.VMEM((EPW, CH), jnp.int32)]
            + [pltpu.VMEM((CH, 2 * ED), jnp.float32) for _ in range(NBUF)]
            + [pltpu.SemaphoreType.DMA for _ in range(2 * NBUF)]
        ),
    )
    def sc_kv(pkv, adj_i, kvg, *scratch):
        idx_all = scratch[0]
        rows = scratch[1:1 + NBUF]
        sem_g = scratch[1 + NBUF:1 + 2 * NBUF]
        sem_w = scratch[1 + 2 * NBUF:]
        w = lax.axis_index("c") * NS + lax.axis_index("s")
        lo = w * EPW
        cnt = jnp.clip(NCH - lo, 0, EPW)
        pltpu.sync_copy(adj_i.at[pl.ds(lo, EPW)], idx_all)
        ngrp = EPW // NBUF
        kmax = ngrp * NBUF

        def grp(it, carry):
            for b in range(NBUF):
                k = it * NBUF + b

                @pl.when(jnp.logical_and(k >= NBUF, k - NBUF < cnt))
                def _():
                    pltpu.make_async_copy(
                        kvg.at[pl.ds(0, CH)], rows[b], sem_w[b]).wait()

                @pl.when(k < cnt)
                def _():
                    pltpu.async_copy(pkv.at[idx_all.at[k]], rows[b], sem_g[b])
            for b in range(NBUF):
                k = it * NBUF + b

                @pl.when(k < cnt)
                def _():
                    pltpu.make_async_copy(
                        pkv.at[idx_all.at[k]], rows[b], sem_g[b]).wait()
                    pltpu.async_copy(
                        rows[b], kvg.at[pl.ds((lo + k) * CH, CH)], sem_w[b])
            return carry

        lax.fori_loop(0, ngrp, grp, 0)
        for b in range(NBUF):
            @pl.when(kmax - NBUF + b < cnt)
            def _():
                pltpu.make_async_copy(
                    kvg.at[pl.ds(0, CH)], rows[b], sem_w[b]).wait()

    return sc_kv


_sc_kv = _make_sc_kv()


def _make_sc_mean_q():
    mesh = plsc.VectorSubcoreMesh(core_axis_name="c", subcore_axis_name="s")

    @functools.partial(
        pl.kernel,
        mesh=mesh,
        out_type=[
            jax.ShapeDtypeStruct((B, ED), jnp.float32),
            jax.ShapeDtypeStruct((B, ED), jnp.float32),
        ],
        scratch_types=(
            [pltpu.VMEM((EPW, CH), jnp.int32),
             pltpu.VMEM((QPW, QCH), jnp.int32),
             pltpu.VMEM((NBUF, CH), jnp.int32),
             pltpu.VMEM((GB, ED), jnp.float32),
             pltpu.VMEM((GB, ED), jnp.float32)]
            + [pltpu.VMEM((CH, ED), jnp.float32) for _ in range(NBUF)]
            + [pltpu.VMEM((QCH, ED), jnp.float32) for _ in range(QBUF)]
            + [pltpu.VMEM_SHARED((NS * GB, ED), jnp.float32)]
            + [pltpu.SemaphoreType.DMA for _ in range(NBUF)]
            + [pltpu.SemaphoreType.DMA for _ in range(NBUF)]
            + [pltpu.SemaphoreType.DMA]
            + [pltpu.SemaphoreType.DMA for _ in range(QBUF)]
        ),
        compiler_params=pltpu.CompilerParams(use_tc_tiling_on_sc=False),
    )
    def sc_mean_q(pm, pq, dis_i, nod_i, dst_all, zeros_h, ctx, qg, *scratch):
        idx_all, idx_q, dst_v, zrow, ctxb = scratch[0:5]
        rows_m = scratch[5:5 + NBUF]
        rows_q = scratch[5 + NBUF:5 + NBUF + QBUF]
        acc = scratch[5 + NBUF + QBUF]
        sem_g = scratch[6 + NBUF + QBUF:6 + 2 * NBUF + QBUF]
        sem_a = scratch[6 + 2 * NBUF + QBUF:6 + 3 * NBUF + QBUF]
        sem_cw = scratch[6 + 3 * NBUF + QBUF]
        sem_q = scratch[7 + 3 * NBUF + QBUF:]
        sid = lax.axis_index("s")
        w = lax.axis_index("c") * NS + sid

        lo = w * EPW
        cnt = jnp.clip(NCH - lo, 0, EPW)
        pltpu.sync_copy(dis_i.at[pl.ds(lo, EPW)], idx_all)
        pltpu.sync_copy(dst_all.at[pl.ds(sid * NBUF, NBUF)], dst_v)
        pltpu.sync_copy(zeros_h, zrow)
        pltpu.sync_copy(zrow, acc.at[pl.ds(sid * GB, GB)])
        ngrp = EPW // NBUF

        def grp(it, carry):
            @pl.when(it * NBUF < cnt)
            def _():
                for b in range(NBUF):
                    k = it * NBUF + b
                    pltpu.async_copy(pm.at[idx_all.at[k]], rows_m[b], sem_g[b])
                for b in range(NBUF):
                    k = it * NBUF + b
                    pltpu.make_async_copy(
                        pm.at[idx_all.at[k]], rows_m[b], sem_g[b]).wait()
                    pltpu.async_copy(
                        rows_m[b], acc.at[dst_v.at[b]], sem_a[b], add=True)
                for b in range(NBUF):
                    pltpu.make_async_copy(
                        pm.at[pl.ds(0, CH)], rows_m[b], sem_a[b]).wait()
                @pl.when(it > 0)
                def _():
                    pltpu.make_async_copy(
                        pm.at[pl.ds(0, GB)], ctxb, sem_cw).wait()
                pltpu.sync_copy(acc.at[pl.ds(sid * GB, GB)], ctxb)
                pltpu.sync_copy(zrow, acc.at[pl.ds(sid * GB, GB)])
                pltpu.async_copy(
                    ctxb, ctx.at[pl.ds((lo + it * NBUF) * BPC, GB)], sem_cw)
            return carry

        lax.fori_loop(0, ngrp, grp, 0)

        @pl.when(cnt > 0)
        def _():
            pltpu.make_async_copy(pm.at[pl.ds(0, GB)], ctxb, sem_cw).wait()

        qlo = jnp.minimum(w * QPW, NQCHP - QPW)
        qcnt = jnp.clip(NQCH - w * QPW, 0, QPW)
        pltpu.sync_copy(nod_i.at[pl.ds(qlo, QPW)], idx_q)
        qngrp = QPW // QBUF
        qkmax = qngrp * QBUF

        def qgrp(it, carry):
            for b in range(QBUF):
                k = it * QBUF + b

                @pl.when(jnp.logical_and(k >= QBUF, k - QBUF < qcnt))
                def _():
                    pltpu.make_async_copy(
                        qg.at[pl.ds(0, QCH)], rows_q[b], sem_q[b]).wait()

                @pl.when(k < qcnt)
                def _():
                    pltpu.async_copy(pq.at[idx_q.at[k]], rows_q[b], sem_g[b])
            for b in range(QBUF):
                k = it * QBUF + b

                @pl.when(k < qcnt)
                def _():
                    pltpu.make_async_copy(
                        pq.at[idx_q.at[k]], rows_q[b], sem_g[b]).wait()
                    pltpu.async_copy(
                        rows_q[b], qg.at[pl.ds((qlo + k) * QCH, QCH)], sem_q[b])
            return carry

        lax.fori_loop(0, qngrp, qgrp, 0)
        for b in range(QBUF):
            @pl.when(qkmax - QBUF + b < qcnt)
            def _():
                pltpu.make_async_copy(
                    qg.at[pl.ds(0, QCH)], rows_q[b], sem_q[b]).wait()

    return sc_mean_q


_sc_mean_q = _make_sc_mean_q()


def _attn_body(kvg_ref, ctx_ref, qg_ref, wc_ref, bc_ref, out_ref):
    kv = kvg_ref[:].reshape(BB, S, 2 * ED)
    k3 = kv[:, :, :ED]
    v3 = kv[:, :, ED:]
    q = qg_ref[:]
    logits = jnp.sum(k3 * q[:, None, :], axis=-1)
    m = jnp.max(logits, axis=-1, keepdims=True)
    e = jnp.exp(logits - m)
    attn = e / jnp.sum(e, axis=-1, keepdims=True)
    mix = jnp.sum(v3 * attn[:, :, None], axis=1)
    comb = jnp.concatenate([mix, ctx_ref[:]], axis=-1)
    comb = jnp.tanh(jnp.dot(comb, wc_ref[:], preferred_element_type=jnp.float32)
                    + bc_ref[:])
    nrm = jnp.sqrt(jnp.sum(comb * comb, axis=-1, keepdims=True))
    out_ref[:] = comb / jnp.maximum(nrm, 1e-12)


def _attention(kvg, ctx, qg, wc, bc):
    nblk = B // BB
    return pl.pallas_call(
        _attn_body,
        grid=(nblk,),
        in_specs=[
            pl.BlockSpec((BB * S, 2 * ED), lambda i: (i, 0)),
            pl.BlockSpec((BB, ED), lambda i: (i, 0)),
            pl.BlockSpec((BB, ED), lambda i: (i, 0)),
            pl.BlockSpec((2 * ED, ED), lambda i: (0, 0)),
            pl.BlockSpec((1, ED), lambda i: (0, 0)),
        ],
        out_specs=pl.BlockSpec((BB, ED), lambda i: (i, 0)),
        out_shape=jax.ShapeDtypeStruct((B, ED), jnp.float32),
    )(kvg, ctx, qg, wc, bc)


def kernel(id2feat, nodes, adj_neighs, dis_neighs, Wm_w, Wm_b, Wq_w, Wq_b,
           Wk_w, Wk_b, Wv_w, Wv_b, WC_w, WC_b):
    wkv = jnp.concatenate([Wk_w, Wv_w], axis=1)
    bkv = jnp.concatenate([Wk_b, Wv_b])[None, :]
    pkv, pm, pq = _project(id2feat, wkv, Wm_w, Wq_w, bkv,
                           Wm_b[None, :], Wq_b[None, :])
    epad = jnp.zeros((NCHP - NCH, CH), jnp.int32)
    adj2 = jnp.concatenate([adj_neighs.astype(jnp.int32).reshape(NCH, CH), epad])
    dis2 = jnp.concatenate([dis_neighs.astype(jnp.int32).reshape(NCH, CH), epad])
    nod2 = jnp.concatenate([nodes.astype(jnp.int32).reshape(NQCH, QCH),
                            jnp.zeros((NQCHP - NQCH, QCH), jnp.int32)])
    dst_all = ((jnp.arange(NS * NBUF, dtype=jnp.int32) * BPC)[:, None]
               + (jnp.arange(CH, dtype=jnp.int32) // S)[None, :])
    zeros_h = jnp.zeros((GB, ED), jnp.float32)
    kvg = _sc_kv(pkv, adj2)
    ctx, qg = _sc_mean_q(pm, pq, dis2, nod2, dst_all, zeros_h)
    return _attention(kvg, ctx, qg, WC_w, WC_b[None, :])
